# baseline (speedup 1.0000x reference)
import functools

import jax
import jax.numpy as jnp
from jax import lax
from jax.experimental import pallas as pl
from jax.experimental.pallas import tpu as pltpu
from jax.experimental.pallas import tpu_sc as plsc

B = 1024
L = 20
D = 64
V = 100000

NC = 2
NS = 16
NW = NC * NS
B_PER_W = B // NW
G_PER_W = B_PER_W * L
CHUNK = 128
N_CHUNKS = G_PER_W // CHUNK
IDX_ROWS_PER_W = N_CHUNKS

N_TILE = 1024
N_STEPS = (V + N_TILE - 1) // N_TILE


def _sc_gather_mean_body(ids_hbm, table_hbm, out_hbm, idx_v, rows_v, acc_v, sem):
    wid = lax.axis_index("s") * NC + lax.axis_index("c")

    for j in range(N_CHUNKS):
        pltpu.sync_copy(
            ids_hbm.at[pl.ds(wid * G_PER_W + j * CHUNK, CHUNK)], idx_v.at[j]
        )

    copies = []
    for j in range(N_CHUNKS):
        copies.append(
            pltpu.async_copy(
                table_hbm.at[idx_v.at[j]],
                rows_v.at[pl.ds(j * CHUNK, CHUNK)],
                sem,
            )
        )
    for c in copies:
        c.wait()

    def body(b, carry):
        r0 = b * L
        for d in range(D // 16):
            acc = rows_v[r0, pl.ds(d * 16, 16)]
            for j in range(1, L):
                acc = acc + rows_v[r0 + j, pl.ds(d * 16, 16)]
            acc_v[b, pl.ds(d * 16, 16)] = acc * (1.0 / L)
        return carry

    lax.fori_loop(0, B_PER_W, body, 0)

    pltpu.sync_copy(acc_v, out_hbm.at[pl.ds(wid * B_PER_W, B_PER_W)])


_sc_gather_mean = functools.partial(
    pl.kernel,
    mesh=plsc.VectorSubcoreMesh(core_axis_name="c", subcore_axis_name="s"),
    out_type=jax.ShapeDtypeStruct((B, D), jnp.float32),
    compiler_params=pltpu.CompilerParams(use_tc_tiling_on_sc=False),
    scratch_types=[
        pltpu.VMEM((IDX_ROWS_PER_W, CHUNK), jnp.int32),
        pltpu.VMEM((G_PER_W, D), jnp.float32),
        pltpu.VMEM((B_PER_W, D), jnp.float32),
        pltpu.SemaphoreType.DMA,
    ],
)(_sc_gather_mean_body)


def _mm_body(x_ref, w_ref, o_ref):
    o_ref[...] = lax.dot_general(
        x_ref[...],
        w_ref[...],
        dimension_numbers=(((1,), (1,)), ((), ())),
        preferred_element_type=jnp.float32,
    )


def kernel(context_ids, embedding_table, linear_weight):
    ids = context_ids.astype(jnp.int32).reshape(B * L)
    ctx_mean = _sc_gather_mean(ids, embedding_table)
    logits = pl.pallas_call(
        _mm_body,
        grid=(N_STEPS,),
        in_specs=[
            pl.BlockSpec((B, D), lambda n: (0, 0)),
            pl.BlockSpec((N_TILE, D), lambda n: (n, 0)),
        ],
        out_specs=pl.BlockSpec((B, N_TILE), lambda n: (0, n)),
        out_shape=jax.ShapeDtypeStruct((B, V), jnp.float32),
    )(ctx_mean, linear_weight)
    return logits


# speedup vs baseline: 2.7574x; 2.7574x over previous
import functools

import jax
import jax.numpy as jnp
from jax import lax
from jax.experimental import pallas as pl
from jax.experimental.pallas import tpu as pltpu
from jax.experimental.pallas import tpu_sc as plsc

B = 1024
L = 20
D = 64
V = 100000

NC = 2
NS = 16
NW = NC * NS
D_PER_W = D // NW
B_GROUPS = B // 16

N_TILE = 1024
N_STEPS = (V + N_TILE - 1) // N_TILE


def _sc_gather_mean_body(ids_t_hbm, table_t_hbm, out_hbm, ids_v, row_v, out_v, sem):
    wid = lax.axis_index("s") * NC + lax.axis_index("c")
    d0 = wid * D_PER_W

    pltpu.sync_copy(ids_t_hbm, ids_v)

    for k in range(D_PER_W):
        pltpu.async_copy(table_t_hbm.at[pl.ds((d0 + k) * V, V)], row_v, sem).wait()

        def group(g, carry):
            acc = jnp.zeros((16,), jnp.float32)
            for j in range(L):
                idx = ids_v[j, pl.ds(g * 16, 16)]
                acc = acc + plsc.load_gather(row_v, [idx])
            out_v[pl.ds(k * B + g * 16, 16)] = acc * (1.0 / L)
            return carry

        lax.fori_loop(0, B_GROUPS, group, 0)

    pltpu.sync_copy(out_v, out_hbm.at[pl.ds(d0 * B, D_PER_W * B)])


_sc_gather_mean = functools.partial(
    pl.kernel,
    mesh=plsc.VectorSubcoreMesh(core_axis_name="c", subcore_axis_name="s"),
    out_type=jax.ShapeDtypeStruct((D * B,), jnp.float32),
    compiler_params=pltpu.CompilerParams(needs_layout_passes=False),
    scratch_types=[
        pltpu.VMEM((L, B), jnp.int32),
        pltpu.VMEM((V,), jnp.float32),
        pltpu.VMEM((D_PER_W * B,), jnp.float32),
        pltpu.SemaphoreType.DMA,
    ],
)(_sc_gather_mean_body)


def _mm_body(w_ref, x_ref, o_ref):
    o_ref[...] = lax.dot_general(
        w_ref[...],
        x_ref[...],
        dimension_numbers=(((0,), (0,)), ((), ())),
        preferred_element_type=jnp.float32,
    )


def kernel(context_ids, embedding_table, linear_weight):
    ids_t = context_ids.astype(jnp.int32).T
    table_t = embedding_table.T.reshape(D * V)
    w_t = linear_weight.T
    ctx_mean_t = _sc_gather_mean(ids_t, table_t).reshape(D, B)
    logits_t = pl.pallas_call(
        _mm_body,
        grid=(N_STEPS,),
        in_specs=[
            pl.BlockSpec((D, N_TILE), lambda n: (0, n)),
            pl.BlockSpec((D, B), lambda n: (0, 0)),
        ],
        out_specs=pl.BlockSpec((N_TILE, B), lambda n: (n, 0)),
        out_shape=jax.ShapeDtypeStruct((V, B), jnp.float32),
    )(w_t, ctx_mean_t)
    return logits_t.T


# speedup vs baseline: 3.0643x; 1.1113x over previous
import functools

import jax
import jax.numpy as jnp
from jax import lax
from jax.experimental import pallas as pl
from jax.experimental.pallas import tpu as pltpu
from jax.experimental.pallas import tpu_sc as plsc

B = 1024
L = 20
D = 64
V = 100000

NC = 2
NS = 16
NW = NC * NS
D_PER_W = D // NW
B_GROUPS = B // 16

N_TILE = 2048
N_STEPS = (V + N_TILE - 1) // N_TILE


def _sc_gather_mean_body(ids_t_hbm, table_t_hbm, out_hbm, ids_v, row_v, out_v, sem):
    wid = lax.axis_index("s") * NC + lax.axis_index("c")
    d0 = wid * D_PER_W

    pltpu.sync_copy(ids_t_hbm, ids_v)

    for k in range(D_PER_W):
        pltpu.async_copy(table_t_hbm.at[pl.ds((d0 + k) * V, V)], row_v, sem).wait()

        def group(g, carry):
            acc = jnp.zeros((16,), jnp.float32)
            for j in range(L):
                idx = ids_v[j, pl.ds(g * 16, 16)]
                acc = acc + plsc.load_gather(row_v, [idx])
            out_v[pl.ds(k * B + g * 16, 16)] = acc * (1.0 / L)
            return carry

        lax.fori_loop(0, B_GROUPS, group, 0)

    pltpu.sync_copy(out_v, out_hbm.at[pl.ds(d0 * B, D_PER_W * B)])


_sc_gather_mean = functools.partial(
    pl.kernel,
    mesh=plsc.VectorSubcoreMesh(core_axis_name="c", subcore_axis_name="s"),
    out_type=jax.ShapeDtypeStruct((D * B,), jnp.float32),
    compiler_params=pltpu.CompilerParams(needs_layout_passes=False),
    scratch_types=[
        pltpu.VMEM((L, B), jnp.int32),
        pltpu.VMEM((V,), jnp.float32),
        pltpu.VMEM((D_PER_W * B,), jnp.float32),
        pltpu.SemaphoreType.DMA,
    ],
)(_sc_gather_mean_body)


def _mm_body(w_ref, x_ref, o_ref):
    o_ref[...] = lax.dot_general(
        w_ref[...],
        x_ref[...],
        dimension_numbers=(((0,), (0,)), ((), ())),
        preferred_element_type=jnp.float32,
    )


def kernel(context_ids, embedding_table, linear_weight):
    ids_t = context_ids.astype(jnp.int32).T
    table_t = embedding_table.T.reshape(D * V)
    w_t = linear_weight.T
    ctx_mean_t = _sc_gather_mean(ids_t, table_t).reshape(D, B)
    logits_t = pl.pallas_call(
        _mm_body,
        grid=(N_STEPS,),
        in_specs=[
            pl.BlockSpec((D, N_TILE), lambda n: (0, n)),
            pl.BlockSpec((D, B), lambda n: (0, 0)),
        ],
        out_specs=pl.BlockSpec((N_TILE, B), lambda n: (n, 0)),
        out_shape=jax.ShapeDtypeStruct((V, B), jnp.float32),
    )(w_t, ctx_mean_t)
    return logits_t.T


# speedup vs baseline: 3.0928x; 1.0093x over previous
import functools

import jax
import jax.numpy as jnp
from jax import lax
from jax.experimental import pallas as pl
from jax.experimental.pallas import tpu as pltpu
from jax.experimental.pallas import tpu_sc as plsc

B = 1024
L = 20
D = 64
V = 100000

NC = 2
NS = 16
NW = NC * NS
D_PER_W = D // NW
B_GROUPS = B // 16

N_TILE = 4096
N_STEPS = (V + N_TILE - 1) // N_TILE


def _sc_gather_mean_body(ids_t_hbm, table_t_hbm, out_hbm, ids_v, row_v, out_v, sem):
    wid = lax.axis_index("s") * NC + lax.axis_index("c")
    d0 = wid * D_PER_W

    pltpu.sync_copy(ids_t_hbm, ids_v)

    for k in range(D_PER_W):
        pltpu.async_copy(table_t_hbm.at[pl.ds((d0 + k) * V, V)], row_v, sem).wait()

        def group(g, carry):
            acc = jnp.zeros((16,), jnp.float32)
            for j in range(L):
                idx = ids_v[j, pl.ds(g * 16, 16)]
                acc = acc + plsc.load_gather(row_v, [idx])
            out_v[pl.ds(k * B + g * 16, 16)] = acc * (1.0 / L)
            return carry

        lax.fori_loop(0, B_GROUPS, group, 0)

    pltpu.sync_copy(out_v, out_hbm.at[pl.ds(d0 * B, D_PER_W * B)])


_sc_gather_mean = functools.partial(
    pl.kernel,
    mesh=plsc.VectorSubcoreMesh(core_axis_name="c", subcore_axis_name="s"),
    out_type=jax.ShapeDtypeStruct((D * B,), jnp.float32),
    compiler_params=pltpu.CompilerParams(needs_layout_passes=False),
    scratch_types=[
        pltpu.VMEM((L, B), jnp.int32),
        pltpu.VMEM((V,), jnp.float32),
        pltpu.VMEM((D_PER_W * B,), jnp.float32),
        pltpu.SemaphoreType.DMA,
    ],
)(_sc_gather_mean_body)


def _mm_body(w_ref, x_ref, o_ref):
    o_ref[...] = lax.dot_general(
        w_ref[...],
        x_ref[...],
        dimension_numbers=(((0,), (0,)), ((), ())),
        preferred_element_type=jnp.float32,
    )


def kernel(context_ids, embedding_table, linear_weight):
    ids_t = context_ids.astype(jnp.int32).T
    table_t = embedding_table.T.reshape(D * V)
    w_t = linear_weight.T
    ctx_mean_t = _sc_gather_mean(ids_t, table_t).reshape(D, B)
    logits_t = pl.pallas_call(
        _mm_body,
        grid=(N_STEPS,),
        in_specs=[
            pl.BlockSpec((D, N_TILE), lambda n: (0, n)),
            pl.BlockSpec((D, B), lambda n: (0, 0)),
        ],
        out_specs=pl.BlockSpec((N_TILE, B), lambda n: (n, 0)),
        out_shape=jax.ShapeDtypeStruct((V, B), jnp.float32),
    )(w_t, ctx_mean_t)
    return logits_t.T


# speedup vs baseline: 3.0973x; 1.0014x over previous
import functools

import jax
import jax.numpy as jnp
from jax import lax
from jax.experimental import pallas as pl
from jax.experimental.pallas import tpu as pltpu
from jax.experimental.pallas import tpu_sc as plsc

B = 1024
L = 20
D = 64
V = 100000

NC = 2
NS = 16
NW = NC * NS
D_PER_W = D // NW
B_GROUPS = B // 16

N_TILE = 6144
N_STEPS = (V + N_TILE - 1) // N_TILE


def _sc_gather_mean_body(ids_t_hbm, table_t_hbm, out_hbm, ids_v, row_v, out_v, sem):
    wid = lax.axis_index("s") * NC + lax.axis_index("c")
    d0 = wid * D_PER_W

    pltpu.sync_copy(ids_t_hbm, ids_v)

    for k in range(D_PER_W):
        pltpu.async_copy(table_t_hbm.at[pl.ds((d0 + k) * V, V)], row_v, sem).wait()

        def group(g, carry):
            acc = jnp.zeros((16,), jnp.float32)
            for j in range(L):
                idx = ids_v[j, pl.ds(g * 16, 16)]
                acc = acc + plsc.load_gather(row_v, [idx])
            out_v[pl.ds(k * B + g * 16, 16)] = acc * (1.0 / L)
            return carry

        lax.fori_loop(0, B_GROUPS, group, 0)

    pltpu.sync_copy(out_v, out_hbm.at[pl.ds(d0 * B, D_PER_W * B)])


_sc_gather_mean = functools.partial(
    pl.kernel,
    mesh=plsc.VectorSubcoreMesh(core_axis_name="c", subcore_axis_name="s"),
    out_type=jax.ShapeDtypeStruct((D * B,), jnp.float32),
    compiler_params=pltpu.CompilerParams(needs_layout_passes=False),
    scratch_types=[
        pltpu.VMEM((L, B), jnp.int32),
        pltpu.VMEM((V,), jnp.float32),
        pltpu.VMEM((D_PER_W * B,), jnp.float32),
        pltpu.SemaphoreType.DMA,
    ],
)(_sc_gather_mean_body)


def _mm_body(w_ref, x_ref, o_ref):
    o_ref[...] = lax.dot_general(
        w_ref[...],
        x_ref[...],
        dimension_numbers=(((0,), (0,)), ((), ())),
        preferred_element_type=jnp.float32,
    )


def kernel(context_ids, embedding_table, linear_weight):
    ids_t = context_ids.astype(jnp.int32).T
    table_t = embedding_table.T.reshape(D * V)
    w_t = linear_weight.T
    ctx_mean_t = _sc_gather_mean(ids_t, table_t).reshape(D, B)
    logits_t = pl.pallas_call(
        _mm_body,
        grid=(N_STEPS,),
        in_specs=[
            pl.BlockSpec((D, N_TILE), lambda n: (0, n)),
            pl.BlockSpec((D, B), lambda n: (0, 0)),
        ],
        out_specs=pl.BlockSpec((N_TILE, B), lambda n: (n, 0)),
        out_shape=jax.ShapeDtypeStruct((V, B), jnp.float32),
    )(w_t, ctx_mean_t)
    return logits_t.T
